# baseline (speedup 1.0000x reference)
import functools

import jax
import jax.numpy as jnp
from jax.experimental import pallas as pl
from jax.experimental.pallas import tpu as pltpu


def _fused_kernel(x_ref, w0_ref, b0_ref, w1_ref, b1_ref, d0_ref, db0_ref,
                  d1_ref, db1_ref, ow_ref, ob_ref, o_ref, *, L, tb):
    def mm(w, a):
        return jnp.dot(w, a, preferred_element_type=jnp.float32)

    X = x_ref[0]
    c_in = X.shape[0]

    z0 = jnp.zeros((c_in, tb), jnp.bfloat16)
    hp = jnp.concatenate([z0, X, z0], axis=1)
    a0 = jnp.concatenate([hp[:, 0:L * tb],
                          hp[:, tb:(L + 1) * tb],
                          hp[:, 2 * tb:(L + 2) * tb]], axis=0)
    y = jnp.maximum(mm(w0_ref[...], a0) + b0_ref[...], 0.0)
    y = y.astype(jnp.bfloat16)

    c1 = y.shape[0]
    z1 = jnp.zeros((c1, tb), jnp.bfloat16)
    hp1 = jnp.concatenate([z1, y, z1], axis=1)
    a1 = jnp.concatenate([hp1[:, 0:L * tb],
                          hp1[:, tb:(L + 1) * tb],
                          hp1[:, 2 * tb:(L + 2) * tb]], axis=0)
    y2 = jnp.maximum(mm(w1_ref[...], a1) + b1_ref[...], 0.0)
    y2 = y2.astype(jnp.bfloat16)

    ycat = jnp.concatenate(
        [y2[:, l * tb:(l + 1) * tb] for l in range(L)], axis=0)
    z = jnp.maximum(mm(d0_ref[...], ycat) + db0_ref[...], 0.0)
    z = z.astype(jnp.bfloat16)

    z = jnp.maximum(mm(d1_ref[...], z) + db1_ref[...], 0.0)
    z = z.astype(jnp.bfloat16)
    o_ref[...] = mm(ow_ref[...], z) + ob_ref[...]


def kernel(x, conv0_w, conv0_b, conv1_w, conv1_b,
           dense0_w, dense0_b, dense1_w, dense1_b, out_w, out_b):
    B, c_in, L = x.shape
    c1 = conv1_w.shape[1]
    n_out = out_w.shape[1]

    tb = 512
    while B % tb:
        tb //= 2
    nb = B // tb

    xg = jnp.transpose(x.astype(jnp.bfloat16).reshape(nb, tb, c_in, L),
                       (0, 2, 3, 1))
    xg = xg.reshape(nb, c_in, L * tb)
    w0 = conv0_w.T.astype(jnp.bfloat16)
    w1 = conv1_w.T.astype(jnp.bfloat16)
    d0 = dense0_w.T.astype(jnp.bfloat16)
    d1 = dense1_w.T.astype(jnp.bfloat16)
    ow = out_w.T.astype(jnp.bfloat16)
    b0 = conv0_b.reshape(-1, 1)
    b1 = conv1_b.reshape(-1, 1)
    db0 = dense0_b.reshape(-1, 1)
    db1 = dense1_b.reshape(-1, 1)
    ob = out_b.reshape(-1, 1)

    grid = (nb,)

    def bcast(arr):
        return pl.BlockSpec(arr.shape, lambda b: (0,) * arr.ndim)

    in_specs = [pl.BlockSpec((1, c_in, L * tb), lambda b: (b, 0, 0)),
                bcast(w0), bcast(b0), bcast(w1), bcast(b1),
                bcast(d0), bcast(db0), bcast(d1), bcast(db1),
                bcast(ow), bcast(ob)]
    out_specs = pl.BlockSpec((n_out, tb), lambda b: (0, b))

    flops = 2 * B * L * (3 * c_in * conv0_w.shape[1] + 3 * c1 * c1) \
        + 2 * B * (L * c1 * dense0_w.shape[1]
                   + dense1_w.shape[0] * dense1_w.shape[1]
                   + out_w.shape[0] * n_out)
    weights = [w0, b0, w1, b1, d0, db0, d1, db1, ow, ob]
    param_bytes = sum(int(a.size) * a.dtype.itemsize for a in weights)
    bytes_accessed = int(xg.size) * 2 + param_bytes + B * n_out * 4
    cost = pl.CostEstimate(flops=int(flops), transcendentals=0,
                           bytes_accessed=int(bytes_accessed))

    kern = functools.partial(_fused_kernel, L=L, tb=tb)
    ot = pl.pallas_call(
        kern,
        out_shape=jax.ShapeDtypeStruct((n_out, B), jnp.float32),
        grid=grid,
        in_specs=in_specs,
        out_specs=out_specs,
        compiler_params=pltpu.CompilerParams(
            dimension_semantics=("parallel",),
            vmem_limit_bytes=100 * 1024 * 1024),
        cost_estimate=cost,
    )(xg, w0, b0, w1, b1, d0, db0, d1, db1, ow, ob)
    return ot.T


# speedup vs baseline: 1.4039x; 1.4039x over previous
import functools

import jax
import jax.numpy as jnp
from jax.experimental import pallas as pl
from jax.experimental.pallas import tpu as pltpu


def _fused_kernel(x_ref, w0_ref, b0_ref, w1_ref, b1_ref, d0_ref, db0_ref,
                  d1_ref, db1_ref, ow_ref, ob_ref, o_ref, *, L, tb):
    def mm(w, a):
        return jnp.dot(w, a, preferred_element_type=jnp.float32)

    c_in = x_ref.shape[2]

    h = x_ref[...].astype(jnp.bfloat16).reshape(L * tb, c_in)
    z0 = jnp.zeros((tb, c_in), jnp.bfloat16)
    hp = jnp.concatenate([z0, h, z0], axis=0)
    a0 = jnp.concatenate([hp[0:L * tb],
                          hp[tb:(L + 1) * tb],
                          hp[2 * tb:(L + 2) * tb]], axis=1)
    y = jax.lax.dot_general(w0_ref[...], a0, (((0,), (1,)), ((), ())),
                            preferred_element_type=jnp.float32)
    y = jnp.maximum(y + b0_ref[...], 0.0)
    y = y.astype(jnp.bfloat16)

    c1 = y.shape[0]
    z1 = jnp.zeros((c1, tb), jnp.bfloat16)
    hp1 = jnp.concatenate([z1, y, z1], axis=1)
    a1 = jnp.concatenate([hp1[:, 0:L * tb],
                          hp1[:, tb:(L + 1) * tb],
                          hp1[:, 2 * tb:(L + 2) * tb]], axis=0)
    y2 = jnp.maximum(mm(w1_ref[...], a1) + b1_ref[...], 0.0)
    y2 = y2.astype(jnp.bfloat16)

    ycat = jnp.concatenate(
        [y2[:, l * tb:(l + 1) * tb] for l in range(L)], axis=0)
    z = jnp.maximum(mm(d0_ref[...], ycat) + db0_ref[...], 0.0)
    z = z.astype(jnp.bfloat16)

    z = jnp.maximum(mm(d1_ref[...], z) + db1_ref[...], 0.0)
    z = z.astype(jnp.bfloat16)
    o_ref[...] = mm(ow_ref[...], z) + ob_ref[...]


def kernel(x, conv0_w, conv0_b, conv1_w, conv1_b,
           dense0_w, dense0_b, dense1_w, dense1_b, out_w, out_b):
    B, c_in, L = x.shape
    c1 = conv1_w.shape[1]
    n_out = out_w.shape[1]

    tb = 512
    while B % tb:
        tb //= 2
    nb = B // tb

    xg = jnp.transpose(x, (2, 0, 1))
    w0 = conv0_w.astype(jnp.bfloat16)
    w1 = conv1_w.T.astype(jnp.bfloat16)
    d0 = dense0_w.T.astype(jnp.bfloat16)
    d1 = dense1_w.T.astype(jnp.bfloat16)
    ow = out_w.T.astype(jnp.bfloat16)
    b0 = conv0_b.reshape(-1, 1)
    b1 = conv1_b.reshape(-1, 1)
    db0 = dense0_b.reshape(-1, 1)
    db1 = dense1_b.reshape(-1, 1)
    ob = out_b.reshape(-1, 1)

    grid = (nb,)

    def bcast(arr):
        return pl.BlockSpec(arr.shape, lambda b: (0,) * arr.ndim)

    in_specs = [pl.BlockSpec((L, tb, c_in), lambda b: (0, b, 0)),
                bcast(w0), bcast(b0), bcast(w1), bcast(b1),
                bcast(d0), bcast(db0), bcast(d1), bcast(db1),
                bcast(ow), bcast(ob)]
    out_specs = pl.BlockSpec((n_out, tb), lambda b: (0, b))

    flops = 2 * B * L * (3 * c_in * conv0_w.shape[1] + 3 * c1 * c1) \
        + 2 * B * (L * c1 * dense0_w.shape[1]
                   + dense1_w.shape[0] * dense1_w.shape[1]
                   + out_w.shape[0] * n_out)
    weights = [w0, b0, w1, b1, d0, db0, d1, db1, ow, ob]
    param_bytes = sum(int(a.size) * a.dtype.itemsize for a in weights)
    bytes_accessed = int(xg.size) * 4 + param_bytes + B * n_out * 4
    cost = pl.CostEstimate(flops=int(flops), transcendentals=0,
                           bytes_accessed=int(bytes_accessed))

    kern = functools.partial(_fused_kernel, L=L, tb=tb)
    ot = pl.pallas_call(
        kern,
        out_shape=jax.ShapeDtypeStruct((n_out, B), jnp.float32),
        grid=grid,
        in_specs=in_specs,
        out_specs=out_specs,
        compiler_params=pltpu.CompilerParams(
            dimension_semantics=("parallel",),
            vmem_limit_bytes=100 * 1024 * 1024),
        cost_estimate=cost,
    )(xg, w0, b0, w1, b1, d0, db0, d1, db1, ow, ob)
    return ot.T


# speedup vs baseline: 1.4368x; 1.0234x over previous
import functools

import jax
import jax.numpy as jnp
from jax.experimental import pallas as pl
from jax.experimental.pallas import tpu as pltpu


def _fused_kernel(x_ref, w0_ref, b0_ref, w1_ref, b1_ref, d0_ref, db0_ref,
                  d1_ref, db1_ref, ow_ref, ob_ref, o_ref, *, L, tb):
    def mm(w, a):
        return jnp.dot(w, a, preferred_element_type=jnp.float32)

    c_in = x_ref.shape[2]

    h = x_ref[...].astype(jnp.bfloat16).reshape(L * tb, c_in)
    z0 = jnp.zeros((tb, c_in), jnp.bfloat16)
    hp = jnp.concatenate([z0, h, z0], axis=0)
    a0 = jnp.concatenate([hp[0:L * tb],
                          hp[tb:(L + 1) * tb],
                          hp[2 * tb:(L + 2) * tb]], axis=1)
    y = jax.lax.dot_general(w0_ref[...], a0, (((0,), (1,)), ((), ())),
                            preferred_element_type=jnp.float32)
    y = jnp.maximum(y + b0_ref[...], 0.0)
    y = y.astype(jnp.bfloat16)

    c1 = y.shape[0]
    z1 = jnp.zeros((c1, tb), jnp.bfloat16)
    hp1 = jnp.concatenate([z1, y, z1], axis=1)
    a1 = jnp.concatenate([hp1[:, 0:L * tb],
                          hp1[:, tb:(L + 1) * tb],
                          hp1[:, 2 * tb:(L + 2) * tb]], axis=0)
    y2 = jnp.maximum(mm(w1_ref[...], a1) + b1_ref[...], 0.0)
    y2 = y2.astype(jnp.bfloat16)

    ycat = jnp.concatenate(
        [y2[:, l * tb:(l + 1) * tb] for l in range(L)], axis=0)
    z = jax.lax.dot_general(d0_ref[...], ycat, (((0,), (0,)), ((), ())),
                            preferred_element_type=jnp.float32)
    z = jnp.maximum(z + db0_ref[...], 0.0)
    z = z.astype(jnp.bfloat16)

    z = jnp.maximum(mm(d1_ref[...], z) + db1_ref[...], 0.0)
    z = z.astype(jnp.bfloat16)
    o_ref[...] = jax.lax.dot_general(
        z, ow_ref[...], (((0,), (0,)), ((), ())),
        preferred_element_type=jnp.float32) + ob_ref[...]


def kernel(x, conv0_w, conv0_b, conv1_w, conv1_b,
           dense0_w, dense0_b, dense1_w, dense1_b, out_w, out_b):
    B, c_in, L = x.shape
    c1 = conv1_w.shape[1]
    n_out = out_w.shape[1]

    tb = 512
    while B % tb:
        tb //= 2
    nb = B // tb

    xg = jnp.transpose(x, (2, 0, 1))
    w0 = conv0_w.astype(jnp.bfloat16)
    w1 = conv1_w.T.astype(jnp.bfloat16)
    d0 = dense0_w.astype(jnp.bfloat16)
    d1 = dense1_w.T.astype(jnp.bfloat16)
    ow = out_w.astype(jnp.bfloat16)
    b0 = conv0_b.reshape(-1, 1)
    b1 = conv1_b.reshape(-1, 1)
    db0 = dense0_b.reshape(-1, 1)
    db1 = dense1_b.reshape(-1, 1)
    ob = out_b

    grid = (nb,)

    def bcast(arr):
        return pl.BlockSpec(arr.shape, lambda b: (0,) * arr.ndim)

    in_specs = [pl.BlockSpec((L, tb, c_in), lambda b: (0, b, 0)),
                bcast(w0), bcast(b0), bcast(w1), bcast(b1),
                bcast(d0), bcast(db0), bcast(d1), bcast(db1),
                bcast(ow), bcast(ob)]
    out_specs = pl.BlockSpec((tb, n_out), lambda b: (b, 0))

    flops = 2 * B * L * (3 * c_in * conv0_w.shape[1] + 3 * c1 * c1) \
        + 2 * B * (L * c1 * dense0_w.shape[1]
                   + dense1_w.shape[0] * dense1_w.shape[1]
                   + out_w.shape[0] * n_out)
    weights = [w0, b0, w1, b1, d0, db0, d1, db1, ow, ob]
    param_bytes = sum(int(a.size) * a.dtype.itemsize for a in weights)
    bytes_accessed = int(xg.size) * 4 + param_bytes + B * n_out * 4
    cost = pl.CostEstimate(flops=int(flops), transcendentals=0,
                           bytes_accessed=int(bytes_accessed))

    kern = functools.partial(_fused_kernel, L=L, tb=tb)
    ot = pl.pallas_call(
        kern,
        out_shape=jax.ShapeDtypeStruct((B, n_out), jnp.float32),
        grid=grid,
        in_specs=in_specs,
        out_specs=out_specs,
        compiler_params=pltpu.CompilerParams(
            dimension_semantics=("parallel",),
            vmem_limit_bytes=100 * 1024 * 1024),
        cost_estimate=cost,
    )(xg, w0, b0, w1, b1, d0, db0, d1, db1, ow, ob)
    return ot


# speedup vs baseline: 1.4503x; 1.0094x over previous
import functools

import jax
import jax.numpy as jnp
from jax.experimental import pallas as pl
from jax.experimental.pallas import tpu as pltpu


def _fused_kernel(x_ref, w0_ref, b0_ref, w1_ref, b1_ref, d0_ref, db0_ref,
                  d1_ref, db1_ref, ow_ref, ob_ref, o_ref, *, L, tb):
    def mm(w, a):
        return jnp.dot(w, a, preferred_element_type=jnp.float32)

    c_in = x_ref.shape[2]

    h = x_ref[...].astype(jnp.bfloat16).reshape(L * tb, c_in)
    z0 = jnp.zeros((tb, c_in), jnp.bfloat16)
    hp = jnp.concatenate([z0, h, z0], axis=0)
    a0 = jnp.concatenate([hp[0:L * tb],
                          hp[tb:(L + 1) * tb],
                          hp[2 * tb:(L + 2) * tb]], axis=1)
    y = jax.lax.dot_general(w0_ref[...], a0, (((0,), (1,)), ((), ())),
                            preferred_element_type=jnp.float32)
    y = jnp.maximum(y.astype(jnp.bfloat16) + b0_ref[...], 0)

    c1 = y.shape[0]
    z1 = jnp.zeros((c1, tb), jnp.bfloat16)
    hp1 = jnp.concatenate([z1, y, z1], axis=1)
    a1 = jnp.concatenate([hp1[:, 0:L * tb],
                          hp1[:, tb:(L + 1) * tb],
                          hp1[:, 2 * tb:(L + 2) * tb]], axis=0)
    y2 = jnp.maximum(mm(w1_ref[...], a1).astype(jnp.bfloat16)
                     + b1_ref[...], 0)

    ycat = jnp.concatenate(
        [y2[:, l * tb:(l + 1) * tb] for l in range(L)], axis=0)
    z = jax.lax.dot_general(d0_ref[...], ycat, (((0,), (0,)), ((), ())),
                            preferred_element_type=jnp.float32)
    z = jnp.maximum(z.astype(jnp.bfloat16) + db0_ref[...], 0)

    z = jnp.maximum(mm(d1_ref[...], z).astype(jnp.bfloat16)
                    + db1_ref[...], 0)
    o_ref[...] = jax.lax.dot_general(
        z, ow_ref[...], (((0,), (0,)), ((), ())),
        preferred_element_type=jnp.float32) + ob_ref[...]


def kernel(x, conv0_w, conv0_b, conv1_w, conv1_b,
           dense0_w, dense0_b, dense1_w, dense1_b, out_w, out_b):
    B, c_in, L = x.shape
    c1 = conv1_w.shape[1]
    n_out = out_w.shape[1]

    tb = 512
    while B % tb:
        tb //= 2
    nb = B // tb

    xg = jnp.transpose(x, (2, 0, 1))
    w0 = conv0_w.astype(jnp.bfloat16)
    w1 = conv1_w.T.astype(jnp.bfloat16)
    d0 = dense0_w.astype(jnp.bfloat16)
    d1 = dense1_w.T.astype(jnp.bfloat16)
    ow = out_w.astype(jnp.bfloat16)
    b0 = conv0_b.reshape(-1, 1).astype(jnp.bfloat16)
    b1 = conv1_b.reshape(-1, 1).astype(jnp.bfloat16)
    db0 = dense0_b.reshape(-1, 1).astype(jnp.bfloat16)
    db1 = dense1_b.reshape(-1, 1).astype(jnp.bfloat16)
    ob = out_b

    grid = (nb,)

    def bcast(arr):
        return pl.BlockSpec(arr.shape, lambda b: (0,) * arr.ndim)

    in_specs = [pl.BlockSpec((L, tb, c_in), lambda b: (0, b, 0)),
                bcast(w0), bcast(b0), bcast(w1), bcast(b1),
                bcast(d0), bcast(db0), bcast(d1), bcast(db1),
                bcast(ow), bcast(ob)]
    out_specs = pl.BlockSpec((tb, n_out), lambda b: (b, 0))

    flops = 2 * B * L * (3 * c_in * conv0_w.shape[1] + 3 * c1 * c1) \
        + 2 * B * (L * c1 * dense0_w.shape[1]
                   + dense1_w.shape[0] * dense1_w.shape[1]
                   + out_w.shape[0] * n_out)
    weights = [w0, b0, w1, b1, d0, db0, d1, db1, ow, ob]
    param_bytes = sum(int(a.size) * a.dtype.itemsize for a in weights)
    bytes_accessed = int(xg.size) * 4 + param_bytes + B * n_out * 4
    cost = pl.CostEstimate(flops=int(flops), transcendentals=0,
                           bytes_accessed=int(bytes_accessed))

    kern = functools.partial(_fused_kernel, L=L, tb=tb)
    ot = pl.pallas_call(
        kern,
        out_shape=jax.ShapeDtypeStruct((B, n_out), jnp.float32),
        grid=grid,
        in_specs=in_specs,
        out_specs=out_specs,
        compiler_params=pltpu.CompilerParams(
            dimension_semantics=("parallel",),
            vmem_limit_bytes=58 * 1024 * 1024),
        cost_estimate=cost,
    )(xg, w0, b0, w1, b1, d0, db0, d1, db1, ow, ob)
    return ot
